# Initial kernel scaffold; baseline (speedup 1.0000x reference)
import jax
import jax.numpy as jnp
from jax.experimental import pallas as pl


def kernel(pos_target, pos_decode_bbox_pred, pos_decode_bbox_targets, stride):
    raise NotImplementedError("write your pallas kernel here")


# speedup vs baseline: 1.1878x; 1.1878x over previous
import jax
import jax.numpy as jnp
from jax.experimental import pallas as pl
from jax.experimental.pallas import tpu as pltpu


def _body(trow_ref, tcol_ref, prow_ref, pcol_ref, tgtT_ref, stride_ref,
          out_ref, acc, c_s, m_s, scale_s):
    i = pl.program_id(0)
    j = pl.program_id(1)
    nb_i = pl.num_programs(0)
    nb_j = pl.num_programs(1)
    B = tcol_ref.shape[0]

    @pl.when((i == 0) & (j == 0))
    def _prologue():
        t = trow_ref[:, :]
        c = jnp.min(t)
        c_s[0] = c
        mask = t == c
        m_s[0] = jnp.sum(mask.astype(jnp.int32))
        idx = jax.lax.broadcasted_iota(jnp.int32, t.shape, 1)
        fi = jnp.min(jnp.where(mask, idx, jnp.int32(2**30)))
        sel = (idx == fi).astype(jnp.float32)
        tg = tgtT_ref[:, :]
        dxs = jnp.sum((tg[2:3, :] - tg[0:1, :]) * sel)
        dys = jnp.sum((tg[3:4, :] - tg[1:2, :]) * sel)
        scale_s[0] = jnp.sqrt(dxs * dxs + dys * dys)
        acc[0] = 0.0

    @pl.when(j <= i)
    def _main():
        c = c_s[0]
        pc = pcol_ref[:, :]
        cxc = (pc[:, 0:1] + pc[:, 2:3]) * 0.5
        cyc = (pc[:, 1:2] + pc[:, 3:4]) * 0.5
        pr = prow_ref[:, pl.ds(j * B, B)]
        cxr = (pr[0:1, :] + pr[2:3, :]) * 0.5
        cyr = (pr[1:2, :] + pr[3:4, :]) * 0.5
        mi = (tcol_ref[:, :] == c).astype(jnp.float32)
        mj = (trow_ref[:, pl.ds(j * B, B)] == c).astype(jnp.float32)
        dx = cxc - cxr
        dy = cyc - cyr
        d = jnp.sqrt(dx * dx + dy * dy)
        s = jnp.sum(jnp.sum(d * mj, axis=1, keepdims=True) * mi)
        acc[0] += s * jnp.where(j == i, 1.0, 2.0)

    @pl.when((i == nb_i - 1) & (j == nb_j - 1))
    def _epilogue():
        m = m_s[0]
        total = acc[0]
        denom = (m * (m - 1)).astype(jnp.float32)
        tot = jnp.where(m != 1, total / denom, total)
        res = tot / scale_s[0] / stride_ref[0, 0]
        out_ref[0, 0] = 2.0 / (1.0 + jnp.exp(-res)) - 1.0


def kernel(pos_target, pos_decode_bbox_pred, pos_decode_bbox_targets, stride):
    n = pos_target.shape[0]
    B = 512
    NB = -(-n // B)
    N = NB * B
    pad = N - n

    t32 = pos_target.astype(jnp.int32)
    big = jnp.iinfo(jnp.int32).max
    t_pad = jnp.pad(t32, (0, pad), constant_values=big)
    trow = t_pad.reshape(1, N)
    tcol = t_pad.reshape(N, 1)
    pred_pad = jnp.pad(pos_decode_bbox_pred, ((0, pad), (0, 0)))
    prow = pred_pad.T
    tgtT = jnp.pad(pos_decode_bbox_targets, ((0, pad), (0, 0))).T
    stride_arr = jnp.asarray(stride, jnp.float32).reshape(1, 1)

    out = pl.pallas_call(
        _body,
        grid=(NB, NB),
        in_specs=[
            pl.BlockSpec((1, N), lambda i, j: (0, 0)),
            pl.BlockSpec((B, 1), lambda i, j: (i, 0)),
            pl.BlockSpec((4, N), lambda i, j: (0, 0)),
            pl.BlockSpec((B, 4), lambda i, j: (i, 0)),
            pl.BlockSpec((4, N), lambda i, j: (0, 0)),
            pl.BlockSpec(memory_space=pltpu.SMEM),
        ],
        out_specs=pl.BlockSpec(memory_space=pltpu.SMEM),
        out_shape=jax.ShapeDtypeStruct((1, 1), jnp.float32),
        scratch_shapes=[
            pltpu.SMEM((1,), jnp.float32),
            pltpu.SMEM((1,), jnp.int32),
            pltpu.SMEM((1,), jnp.int32),
            pltpu.SMEM((1,), jnp.float32),
        ],
    )(trow, tcol, prow, pred_pad, tgtT, stride_arr)
    return out[0, 0]


# speedup vs baseline: 1.4829x; 1.2484x over previous
import functools

import jax
import jax.numpy as jnp
from jax import lax
from jax.experimental import pallas as pl
from jax.experimental.pallas import tpu as pltpu
from jax.experimental.pallas import tpu_sc as plsc


_B = 512


def _k1_body(NL, trow_ref, tlanes_ref, tgtT_ref,
             c16_ref, offs_ref, m_ref, scale_ref):
    t = trow_ref[...]
    c = jnp.min(t)
    mask = t == c
    m_ref[0, 0] = jnp.sum(mask.astype(jnp.int32))
    idx = lax.broadcasted_iota(jnp.int32, t.shape, 1)
    fi = jnp.min(jnp.where(mask, idx, jnp.int32(2**30)))
    sel = (idx == fi).astype(jnp.float32)
    tg = tgtT_ref[...]
    dxs = jnp.sum((tg[2:3, :] - tg[0:1, :]) * sel)
    dys = jnp.sum((tg[3:4, :] - tg[1:2, :]) * sel)
    scale_ref[0, 0] = jnp.sqrt(dxs * dxs + dys * dys)
    c16_ref[...] = jnp.zeros((1, 16), jnp.int32) + c
    cts = jnp.sum((tlanes_ref[...] == c).astype(jnp.float32), axis=1,
                  keepdims=True)
    row = lax.broadcasted_iota(jnp.int32, (NL, NL), 0)
    col = lax.broadcasted_iota(jnp.int32, (NL, NL), 1)
    lower = (col < row).astype(jnp.float32)
    offs = jnp.dot(lower, cts, preferred_element_type=jnp.float32)
    offs_ref[...] = offs.astype(jnp.int32)


def _k2_body(NC, Cw, trash_base,
             t_hbm, x1_hbm, y1_hbm, x2_hbm, y2_hbm, c_hbm, offs_hbm,
             xs_hbm, ys_hbm,
             tbuf, x1b, y1b, x2b, y2b, cxb, cyb, cbuf, obuf, idxbuf,
             semx):
    wid = lax.axis_index("s") * NC + lax.axis_index("c")
    base = wid * Cw
    pltpu.sync_copy(t_hbm.at[pl.ds(base, Cw)], tbuf)
    pltpu.sync_copy(x1_hbm.at[pl.ds(base, Cw)], x1b)
    pltpu.sync_copy(y1_hbm.at[pl.ds(base, Cw)], y1b)
    pltpu.sync_copy(x2_hbm.at[pl.ds(base, Cw)], x2b)
    pltpu.sync_copy(y2_hbm.at[pl.ds(base, Cw)], y2b)
    pltpu.sync_copy(c_hbm, cbuf)
    pltpu.sync_copy(offs_hbm.at[pl.ds(wid * 16, 16)], obuf)
    c_v = cbuf[...]
    run = obuf[...]
    lane = lax.broadcasted_iota(jnp.int32, (16,), 0)
    trash = lane + (trash_base + wid * 16)
    one = jnp.zeros((16,), jnp.int32) + 1
    for g in range(Cw // 16):
        sl = pl.ds(g * 16, 16)
        t_g = tbuf[sl]
        mi = one - jnp.minimum(jnp.abs(t_g - c_v), one)
        fidx = mi * run + (one - mi) * trash
        idxbuf[g // 8, pl.ds((g % 8) * 16, 16)] = fidx
        cxb[sl] = (x1b[sl] + x2b[sl]) * 0.5
        cyb[sl] = (y1b[sl] + y2b[sl]) * 0.5
        run = run + mi
    descs = []
    for b in range(Cw // 128):
        vs = pl.ds(b * 128, 128)
        descs.append(pltpu.async_copy(cxb.at[vs], xs_hbm.at[idxbuf.at[b]], semx))
        descs.append(pltpu.async_copy(cyb.at[vs], ys_hbm.at[idxbuf.at[b]], semx))
    for d in descs:
        d.wait()


def _k3_body(NB, xr_ref, yr_ref, xc_ref, yc_ref, m_ref, scale_ref, stride_ref,
             out_ref, acc):
    i = pl.program_id(0)
    m = m_ref[0, 0]

    @pl.when(i == 0)
    def _init():
        acc[0] = 0.0

    @pl.when(i * _B < m)
    def _main():
        riota = lax.broadcasted_iota(jnp.int32, (_B, 1), 0) + i * _B
        vrow = riota < m
        xc = jnp.where(vrow, xc_ref[...], 0.0)
        yc = jnp.where(vrow, yc_ref[...], 0.0)
        vrowf = vrow.astype(jnp.float32)

        def jbody(j, accum):
            colbase = pl.multiple_of(j * _B, _B)
            ciota = lax.broadcasted_iota(jnp.int32, (1, _B), 1) + colbase
            vcol = ciota < m
            xr = jnp.where(vcol, xr_ref[:, pl.ds(colbase, _B)], 0.0)
            yr = jnp.where(vcol, yr_ref[:, pl.ds(colbase, _B)], 0.0)
            dx = xc - xr
            dy = yc - yr
            d = jnp.sqrt(dx * dx + dy * dy)
            s = jnp.sum(jnp.sum(d * vcol.astype(jnp.float32), axis=1,
                                keepdims=True) * vrowf)
            return accum + s * jnp.where(j == i, 1.0, 2.0)

        acc[0] += lax.fori_loop(0, i + 1, jbody, 0.0)

    @pl.when(i == NB - 1)
    def _epilogue():
        total = acc[0]
        denom = (m * (m - 1)).astype(jnp.float32)
        tot = jnp.where(m != 1, total / denom, total)
        res = tot / scale_ref[0, 0] / stride_ref[0, 0]
        out_ref[0, 0] = 2.0 / (1.0 + jnp.exp(-res)) - 1.0


def kernel(pos_target, pos_decode_bbox_pred, pos_decode_bbox_targets, stride):
    n = pos_target.shape[0]
    info = plsc.get_sparse_core_info()
    NC, NS = info.num_cores, info.num_subcores
    NW = NC * NS
    step = max(_B, NW * 128)
    N = -(-n // step) * step
    pad = N - n
    Cw = N // NW
    NB = N // _B

    t32 = pos_target.astype(jnp.int32)
    big = jnp.iinfo(jnp.int32).max
    t_pad = jnp.pad(t32, (0, pad), constant_values=big)
    pred_pad = jnp.pad(pos_decode_bbox_pred, ((0, pad), (0, 0)))
    tgtT = jnp.pad(pos_decode_bbox_targets, ((0, pad), (0, 0))).T
    stride_arr = jnp.asarray(stride, jnp.float32).reshape(1, 1)

    NL = NW * 16
    G = Cw // 16
    tlanes = t_pad.reshape(NW, G, 16).transpose(0, 2, 1).reshape(NL, G)
    c16, offs, m_arr, scale_arr = pl.pallas_call(
        functools.partial(_k1_body, NL),
        in_specs=[
            pl.BlockSpec((1, N), lambda: (0, 0)),
            pl.BlockSpec((NL, G), lambda: (0, 0)),
            pl.BlockSpec((4, N), lambda: (0, 0)),
        ],
        out_specs=[
            pl.BlockSpec((1, 16), lambda: (0, 0)),
            pl.BlockSpec((NL, 1), lambda: (0, 0)),
            pl.BlockSpec(memory_space=pltpu.SMEM),
            pl.BlockSpec(memory_space=pltpu.SMEM),
        ],
        out_shape=[
            jax.ShapeDtypeStruct((1, 16), jnp.int32),
            jax.ShapeDtypeStruct((NL, 1), jnp.int32),
            jax.ShapeDtypeStruct((1, 1), jnp.int32),
            jax.ShapeDtypeStruct((1, 1), jnp.float32),
        ],
    )(t_pad.reshape(1, N), tlanes, tgtT)

    nbk = Cw // 128
    mesh = plsc.VectorSubcoreMesh(core_axis_name="c", subcore_axis_name="s")
    k2 = pl.kernel(
        functools.partial(_k2_body, NC, Cw, N),
        out_type=[
            jax.ShapeDtypeStruct((N + NL,), jnp.float32),
            jax.ShapeDtypeStruct((N + NL,), jnp.float32),
        ],
        mesh=mesh,
        scratch_types=[
            pltpu.VMEM((Cw,), jnp.int32),
            pltpu.VMEM((Cw,), jnp.float32),
            pltpu.VMEM((Cw,), jnp.float32),
            pltpu.VMEM((Cw,), jnp.float32),
            pltpu.VMEM((Cw,), jnp.float32),
            pltpu.VMEM((Cw,), jnp.float32),
            pltpu.VMEM((Cw,), jnp.float32),
            pltpu.VMEM((16,), jnp.int32),
            pltpu.VMEM((16,), jnp.int32),
            pltpu.VMEM((nbk, 128), jnp.int32),
            pltpu.SemaphoreType.DMA,
        ],
    )
    xs, ys = k2(t_pad, pred_pad[:, 0], pred_pad[:, 1], pred_pad[:, 2],
                pred_pad[:, 3], c16.reshape(16), offs.reshape(NL))

    out = pl.pallas_call(
        functools.partial(_k3_body, NB),
        grid=(NB,),
        in_specs=[
            pl.BlockSpec((1, N), lambda i: (0, 0)),
            pl.BlockSpec((1, N), lambda i: (0, 0)),
            pl.BlockSpec((_B, 1), lambda i: (i, 0)),
            pl.BlockSpec((_B, 1), lambda i: (i, 0)),
            pl.BlockSpec(memory_space=pltpu.SMEM),
            pl.BlockSpec(memory_space=pltpu.SMEM),
            pl.BlockSpec(memory_space=pltpu.SMEM),
        ],
        out_specs=pl.BlockSpec(memory_space=pltpu.SMEM),
        out_shape=jax.ShapeDtypeStruct((1, 1), jnp.float32),
        scratch_shapes=[pltpu.SMEM((1,), jnp.float32)],
    )(xs[:N].reshape(1, N), ys[:N].reshape(1, N),
      xs[:N].reshape(N, 1), ys[:N].reshape(N, 1),
      m_arr, scale_arr, stride_arr)
    return out[0, 0]


# speedup vs baseline: 8.4815x; 5.7196x over previous
import functools

import jax
import jax.numpy as jnp
from jax import lax
from jax.experimental import pallas as pl
from jax.experimental.pallas import tpu as pltpu
from jax.experimental.pallas import tpu_sc as plsc


_B = 512


def _k1_body(NL, NC, trow_ref, tlanes_ref, tgtT_ref,
             c16_ref, offs_ref, m_ref, m0_ref, m1_ref, scale_ref):
    t = trow_ref[...]
    c = jnp.min(t)
    mask = t == c
    m = jnp.sum(mask.astype(jnp.int32))
    m_ref[0, 0] = m
    idx = lax.broadcasted_iota(jnp.int32, t.shape, 1)
    fi = jnp.min(jnp.where(mask, idx, jnp.int32(2**30)))
    sel = (idx == fi).astype(jnp.float32)
    tg = tgtT_ref[...]
    dxs = jnp.sum((tg[2:3, :] - tg[0:1, :]) * sel)
    dys = jnp.sum((tg[3:4, :] - tg[1:2, :]) * sel)
    scale_ref[0, 0] = jnp.sqrt(dxs * dxs + dys * dys)
    c16_ref[...] = jnp.zeros((1, 16), jnp.int32) + c
    cts = jnp.sum((tlanes_ref[...] == c).astype(jnp.float32), axis=1,
                  keepdims=True)
    row = lax.broadcasted_iota(jnp.int32, (NL, NL), 0)
    col = lax.broadcasted_iota(jnp.int32, (NL, NL), 1)
    H = NL // NC
    lower = ((col < row) & ((col // H) == (row // H))).astype(jnp.float32)
    offs = jnp.dot(lower, cts, preferred_element_type=jnp.float32)
    offs_ref[...] = offs.astype(jnp.int32)
    riota = lax.broadcasted_iota(jnp.int32, (NL, 1), 0)
    m0 = jnp.sum(cts * (riota < H).astype(jnp.float32))
    m0_ref[0, 0] = m0.astype(jnp.int32)
    m1_ref[0, 0] = m - m0.astype(jnp.int32)


def _k2_body(NS, Cw, SEG,
             t_hbm, x1_hbm, y1_hbm, x2_hbm, y2_hbm, c_hbm, offs_hbm,
             xs_hbm, ys_hbm,
             tbuf, x1b, y1b, x2b, y2b, cxb, cyb, cbuf, obuf, idxbuf,
             sxs, sys_, semx):
    core = lax.axis_index("c")
    sub = lax.axis_index("s")
    wid = core * NS + sub
    base = wid * Cw
    pltpu.sync_copy(t_hbm.at[pl.ds(base, Cw)], tbuf)
    pltpu.sync_copy(x1_hbm.at[pl.ds(base, Cw)], x1b)
    pltpu.sync_copy(y1_hbm.at[pl.ds(base, Cw)], y1b)
    pltpu.sync_copy(x2_hbm.at[pl.ds(base, Cw)], x2b)
    pltpu.sync_copy(y2_hbm.at[pl.ds(base, Cw)], y2b)
    pltpu.sync_copy(c_hbm, cbuf)
    pltpu.sync_copy(offs_hbm.at[pl.ds(wid * 16, 16)], obuf)
    c_v = cbuf[...]
    run = obuf[...]
    lane = lax.broadcasted_iota(jnp.int32, (16,), 0)
    trash = lane + (NS * Cw + sub * 16)
    one = jnp.zeros((16,), jnp.int32) + 1
    for g in range(Cw // 16):
        sl = pl.ds(g * 16, 16)
        t_g = tbuf[sl]
        mi = one - jnp.minimum(jnp.abs(t_g - c_v), one)
        fidx = mi * run + (one - mi) * trash
        idxbuf[g // 8, pl.ds((g % 8) * 16, 16)] = fidx
        cxb[sl] = (x1b[sl] + x2b[sl]) * 0.5
        cyb[sl] = (y1b[sl] + y2b[sl]) * 0.5
        run = run + mi
    descs = []
    for b in range(Cw // 128):
        vs = pl.ds(b * 128, 128)
        descs.append(pltpu.async_copy(cxb.at[vs], sxs.at[idxbuf.at[b]], semx))
        descs.append(pltpu.async_copy(cyb.at[vs], sys_.at[idxbuf.at[b]], semx))
    for d in descs:
        d.wait()
    plsc.subcore_barrier()

    @pl.when(sub == 0)
    def _flush():
        pltpu.sync_copy(sxs, xs_hbm.at[pl.ds(core * SEG, SEG)])
        pltpu.sync_copy(sys_, ys_hbm.at[pl.ds(core * SEG, SEG)])


def _k3_body(NB, SEGB, xr_ref, yr_ref, xc_ref, yc_ref,
             m_ref, m0_ref, m1_ref, scale_ref, stride_ref, out_ref, acc):
    i = pl.program_id(0)
    m = m_ref[0, 0]
    m0 = m0_ref[0, 0]
    m1 = m1_ref[0, 0]
    segc = SEGB * _B

    @pl.when(i == 0)
    def _init():
        acc[0] = 0.0

    row_active = (i * _B < m0) | ((i >= SEGB) & (i * _B < segc + m1))

    @pl.when(row_active)
    def _main():
        riota = lax.broadcasted_iota(jnp.int32, (_B, 1), 0) + i * _B
        vrow = (riota < m0) | ((riota >= segc) & (riota < segc + m1))
        xc = jnp.where(vrow, xc_ref[...], 0.0)
        yc = jnp.where(vrow, yc_ref[...], 0.0)
        vrowf = vrow.astype(jnp.float32)

        def jbody(j, accum):
            colbase = pl.multiple_of(j * _B, _B)
            ciota = lax.broadcasted_iota(jnp.int32, (1, _B), 1) + colbase
            vcol = (ciota < m0) | ((ciota >= segc) & (ciota < segc + m1))
            xr = jnp.where(vcol, xr_ref[:, pl.ds(colbase, _B)], 0.0)
            yr = jnp.where(vcol, yr_ref[:, pl.ds(colbase, _B)], 0.0)
            dx = xc - xr
            dy = yc - yr
            d = jnp.sqrt(dx * dx + dy * dy)
            s = jnp.sum(jnp.sum(d * vcol.astype(jnp.float32), axis=1,
                                keepdims=True) * vrowf)
            return accum + s * jnp.where(j == i, 1.0, 2.0)

        nb0 = jnp.minimum(i + 1, (m0 + _B - 1) // _B)
        tot = lax.fori_loop(0, nb0, jbody, 0.0)
        tot = lax.fori_loop(SEGB, i + 1, jbody, tot)
        acc[0] += tot

    @pl.when(i == NB - 1)
    def _epilogue():
        total = acc[0]
        denom = (m * (m - 1)).astype(jnp.float32)
        tot = jnp.where(m != 1, total / denom, total)
        res = tot / scale_ref[0, 0] / stride_ref[0, 0]
        out_ref[0, 0] = 2.0 / (1.0 + jnp.exp(-res)) - 1.0


def kernel(pos_target, pos_decode_bbox_pred, pos_decode_bbox_targets, stride):
    n = pos_target.shape[0]
    info = plsc.get_sparse_core_info()
    NC, NS = info.num_cores, info.num_subcores
    NW = NC * NS
    step = max(_B, NW * 128)
    N = -(-n // step) * step
    pad = N - n
    Cw = N // NW
    NL = NW * 16
    G = Cw // 16
    SEG = NS * Cw + _B
    SEGB = SEG // _B
    N3 = NC * SEG
    NB = N3 // _B

    t32 = pos_target.astype(jnp.int32)
    big = jnp.iinfo(jnp.int32).max
    t_pad = jnp.pad(t32, (0, pad), constant_values=big)
    pred_pad = jnp.pad(pos_decode_bbox_pred, ((0, pad), (0, 0)))
    tgtT = jnp.pad(pos_decode_bbox_targets, ((0, pad), (0, 0))).T
    stride_arr = jnp.asarray(stride, jnp.float32).reshape(1, 1)
    tlanes = t_pad.reshape(NW, G, 16).transpose(0, 2, 1).reshape(NL, G)

    c16, offs, m_arr, m0_arr, m1_arr, scale_arr = pl.pallas_call(
        functools.partial(_k1_body, NL, NC),
        in_specs=[
            pl.BlockSpec((1, N), lambda: (0, 0)),
            pl.BlockSpec((NL, G), lambda: (0, 0)),
            pl.BlockSpec((4, N), lambda: (0, 0)),
        ],
        out_specs=[
            pl.BlockSpec((1, 16), lambda: (0, 0)),
            pl.BlockSpec((NL, 1), lambda: (0, 0)),
            pl.BlockSpec(memory_space=pltpu.SMEM),
            pl.BlockSpec(memory_space=pltpu.SMEM),
            pl.BlockSpec(memory_space=pltpu.SMEM),
            pl.BlockSpec(memory_space=pltpu.SMEM),
        ],
        out_shape=[
            jax.ShapeDtypeStruct((1, 16), jnp.int32),
            jax.ShapeDtypeStruct((NL, 1), jnp.int32),
            jax.ShapeDtypeStruct((1, 1), jnp.int32),
            jax.ShapeDtypeStruct((1, 1), jnp.int32),
            jax.ShapeDtypeStruct((1, 1), jnp.int32),
            jax.ShapeDtypeStruct((1, 1), jnp.float32),
        ],
    )(t_pad.reshape(1, N), tlanes, tgtT)

    nbk = Cw // 128
    mesh = plsc.VectorSubcoreMesh(core_axis_name="c", subcore_axis_name="s")
    k2 = pl.kernel(
        functools.partial(_k2_body, NS, Cw, SEG),
        out_type=[
            jax.ShapeDtypeStruct((N3,), jnp.float32),
            jax.ShapeDtypeStruct((N3,), jnp.float32),
        ],
        mesh=mesh,
        scratch_types=[
            pltpu.VMEM((Cw,), jnp.int32),
            pltpu.VMEM((Cw,), jnp.float32),
            pltpu.VMEM((Cw,), jnp.float32),
            pltpu.VMEM((Cw,), jnp.float32),
            pltpu.VMEM((Cw,), jnp.float32),
            pltpu.VMEM((Cw,), jnp.float32),
            pltpu.VMEM((Cw,), jnp.float32),
            pltpu.VMEM((16,), jnp.int32),
            pltpu.VMEM((16,), jnp.int32),
            pltpu.VMEM((nbk, 128), jnp.int32),
            pltpu.VMEM_SHARED((SEG,), jnp.float32),
            pltpu.VMEM_SHARED((SEG,), jnp.float32),
            pltpu.SemaphoreType.DMA,
        ],
    )
    xs, ys = k2(t_pad, pred_pad[:, 0], pred_pad[:, 1], pred_pad[:, 2],
                pred_pad[:, 3], c16.reshape(16), offs.reshape(NL))

    out = pl.pallas_call(
        functools.partial(_k3_body, NB, SEGB),
        grid=(NB,),
        in_specs=[
            pl.BlockSpec((1, N3), lambda i: (0, 0)),
            pl.BlockSpec((1, N3), lambda i: (0, 0)),
            pl.BlockSpec((_B, 1), lambda i: (i, 0)),
            pl.BlockSpec((_B, 1), lambda i: (i, 0)),
            pl.BlockSpec(memory_space=pltpu.SMEM),
            pl.BlockSpec(memory_space=pltpu.SMEM),
            pl.BlockSpec(memory_space=pltpu.SMEM),
            pl.BlockSpec(memory_space=pltpu.SMEM),
            pl.BlockSpec(memory_space=pltpu.SMEM),
        ],
        out_specs=pl.BlockSpec(memory_space=pltpu.SMEM),
        out_shape=jax.ShapeDtypeStruct((1, 1), jnp.float32),
        scratch_shapes=[pltpu.SMEM((1,), jnp.float32)],
    )(xs.reshape(1, N3), ys.reshape(1, N3),
      xs.reshape(N3, 1), ys.reshape(N3, 1),
      m_arr, m0_arr, m1_arr, scale_arr, stride_arr)
    return out[0, 0]
